# baseline (speedup 1.0000x reference)
import functools

import jax
import jax.numpy as jnp
from jax import lax
from jax.experimental import pallas as pl
from jax.experimental.pallas import tpu as pltpu
from jax.experimental.pallas import tpu_sc as plsc

VOCAB = 1000000
EMBED_DIM = 64
BATCH = 4096
HIST = 200

NUM_CORES = 2
NUM_SUBCORES = 16
NUM_WORKERS = NUM_CORES * NUM_SUBCORES
ROWS_PER_WORKER = BATCH // NUM_WORKERS
LANES = 16
NBUF = 4
SPLIT = 128
GROUPS = ROWS_PER_WORKER // NBUF
ACC_UNROLL = 8
ACC_ITERS = HIST // ACC_UNROLL


def _body(ids_hbm, table_hbm, out_hbm, ids_v, rows_v, out_v, s0, s1, s2, s3):
  sems = (s0, s1, s2, s3)
  wid = lax.axis_index("s") * NUM_CORES + lax.axis_index("c")
  base = wid * ROWS_PER_WORKER

  pltpu.sync_copy(ids_hbm.at[pl.ds(base, ROWS_PER_WORKER)], ids_v)

  def issue(b, s):
    pltpu.async_copy(
        table_hbm.at[ids_v.at[b, pl.ds(0, SPLIT)]],
        rows_v.at[s, pl.ds(0, SPLIT)],
        sems[s],
    )
    pltpu.async_copy(
        table_hbm.at[ids_v.at[b, pl.ds(SPLIT, HIST - SPLIT)]],
        rows_v.at[s, pl.ds(SPLIT, HIST - SPLIT)],
        sems[s],
    )

  def wait(s):
    pltpu.make_async_copy(
        table_hbm.at[pl.ds(0, HIST)], rows_v.at[s], sems[s]
    ).wait()

  for s in range(NBUF):
    issue(s, s)

  inv = jnp.float32(1.0 / HIST)

  def group(g, carry):
    for s in range(NBUF):
      b = g * NBUF + s
      wait(s)

      def acc_body(i, acc):
        for j in range(ACC_UNROLL):
          l = i * ACC_UNROLL + j
          acc = tuple(
              acc[k] + rows_v[s, l, pl.ds(k * LANES, LANES)] for k in range(4)
          )
        return acc

      zero = jnp.zeros((LANES,), jnp.float32)
      acc = lax.fori_loop(0, ACC_ITERS, acc_body, (zero, zero, zero, zero))
      for k in range(4):
        out_v[b, pl.ds(k * LANES, LANES)] = acc[k] * inv

      @pl.when(g < GROUPS - 1)
      def _():
        issue(b + NBUF, s)
    return carry

  lax.fori_loop(0, GROUPS, group, 0)
  pltpu.sync_copy(out_v, out_hbm.at[pl.ds(base, ROWS_PER_WORKER)])


@jax.jit
def kernel(ids, table):
  mesh = plsc.VectorSubcoreMesh(
      core_axis_name="c",
      subcore_axis_name="s",
      num_cores=NUM_CORES,
      num_subcores=NUM_SUBCORES,
  )
  run = functools.partial(
      pl.kernel,
      mesh=mesh,
      compiler_params=pltpu.CompilerParams(use_tc_tiling_on_sc=False),
      out_type=jax.ShapeDtypeStruct((BATCH, EMBED_DIM), jnp.float32),
      scratch_types=[
          pltpu.VMEM((ROWS_PER_WORKER, HIST), jnp.int32),
          pltpu.VMEM((NBUF, HIST, EMBED_DIM), jnp.float32),
          pltpu.VMEM((ROWS_PER_WORKER, EMBED_DIM), jnp.float32),
          pltpu.SemaphoreType.DMA,
          pltpu.SemaphoreType.DMA,
          pltpu.SemaphoreType.DMA,
          pltpu.SemaphoreType.DMA,
      ],
  )(_body)
  return run(ids, table)
